# Initial kernel scaffold; baseline (speedup 1.0000x reference)
import jax
import jax.numpy as jnp
from jax.experimental import pallas as pl


def kernel(sequences, seq_lens, emb):
    raise NotImplementedError("write your pallas kernel here")


# speedup vs baseline: 8.0820x; 8.0820x over previous
import functools

import numpy as np
import jax
import jax.numpy as jnp
from jax import lax
from jax.experimental import pallas as pl
from jax.experimental.pallas import tpu as pltpu
from jax.experimental.pallas import tpu_sc as plsc

_B = 1024
_L = 200
_VOCAB = 100000
_D = 64
_MASK_ID = _VOCAB
_GAMMA = 0.7
_TAO = 0.2

_KA = 96
_KB = 48
_CHUNK = 128
_NCORES = 2
_NSUB = 16
_NTILES = _NCORES * _NSUB
_SPT = _B // _NTILES
_ACC_ROWS = _NSUB * _SPT + 8
_TRASH = _NSUB * _SPT

_consts = {}


def _tf_round(x0, x1, r):
    x0 = (x0 + x1) & np.uint64(0xFFFFFFFF)
    x1 = ((x1 << np.uint64(r)) | (x1 >> np.uint64(32 - r))) & np.uint64(0xFFFFFFFF)
    x1 = x0 ^ x1
    return x0, x1


def _threefry2x32(k1, k2, x0, x1):
    rot0 = (13, 15, 26, 6)
    rot1 = (17, 29, 16, 24)
    M = np.uint64(0xFFFFFFFF)
    ks0, ks1 = np.uint64(k1), np.uint64(k2)
    ks2 = ks0 ^ ks1 ^ np.uint64(0x1BD11BDA)
    x0 = (x0 + ks0) & M
    x1 = (x1 + ks1) & M
    for r in rot0:
        x0, x1 = _tf_round(x0, x1, r)
    x0 = (x0 + ks1) & M
    x1 = (x1 + ks2 + np.uint64(1)) & M
    for r in rot1:
        x0, x1 = _tf_round(x0, x1, r)
    x0 = (x0 + ks2) & M
    x1 = (x1 + ks0 + np.uint64(2)) & M
    for r in rot0:
        x0, x1 = _tf_round(x0, x1, r)
    x0 = (x0 + ks0) & M
    x1 = (x1 + ks1 + np.uint64(3)) & M
    for r in rot1:
        x0, x1 = _tf_round(x0, x1, r)
    x0 = (x0 + ks1) & M
    x1 = (x1 + ks2 + np.uint64(4)) & M
    for r in rot0:
        x0, x1 = _tf_round(x0, x1, r)
    x0 = (x0 + ks2) & M
    x1 = (x1 + ks0 + np.uint64(5)) & M
    return x0, x1


def _np_uniform(k1, k2, shape):
    n = int(np.prod(shape))
    b1, b2 = _threefry2x32(k1, k2, np.zeros(n, np.uint64),
                           np.arange(n, dtype=np.uint64))
    bits = (b1 ^ b2).astype(np.uint32)
    fb = (bits >> np.uint32(9)) | np.uint32(0x3F800000)
    return (fb.view(np.float32) - np.float32(1.0)).reshape(shape)


def _aug_consts():
    if "P" not in _consts:
        b1, b2 = _threefry2x32(np.uint64(0), np.uint64(42),
                               np.zeros(2, np.uint64),
                               np.arange(2, dtype=np.uint64))
        bern = _np_uniform(b1[0], b2[0], (_B, _L)) < np.float32(_GAMMA)
        u = _np_uniform(b1[1], b2[1], (_B,))
        keep = ~bern
        cumkeep = np.cumsum(keep, axis=1).astype(np.int32)
        P = np.zeros((_B, _KA), np.int32)
        for b in range(_B):
            pos = np.nonzero(keep[b])[0]
            P[b, : len(pos)] = pos
        _consts["P"] = jnp.asarray(P)
        _consts["cumkeep"] = jnp.asarray(cumkeep)
        _consts["u"] = jnp.asarray(u.astype(np.float32))
    return _consts["P"], _consts["cumkeep"], _consts["u"]


_NCHA = _SPT * _KA // _CHUNK
_NCHB = _SPT * _KB // _CHUNK


def _sc_bag_body(idsA_hbm, segA_hbm, idsB_hbm, segB_hbm, emb_hbm,
                 sumA_hbm, sumB_hbm,
                 idsA_v, segA_v, idsB_v, segB_v,
                 rows0, rows1, accA, accB, sem0, sem1):
    c = lax.axis_index("c")
    s = lax.axis_index("s")
    g = c * _NSUB + s
    base = g * _SPT

    zv = jnp.zeros((16,), jnp.float32)

    def zero_row(i, carry):
        for k in range(_D // 16):
            rows0[i, pl.ds(k * 16, 16)] = zv
        return carry

    lax.fori_loop(0, _SPT, zero_row, 0)
    pltpu.sync_copy(rows0.at[pl.ds(0, _SPT)], accA.at[pl.ds(s * _SPT, _SPT)])
    pltpu.sync_copy(rows0.at[pl.ds(0, _SPT)], accB.at[pl.ds(s * _SPT, _SPT)])

    @pl.when(s == 0)
    def _():
        pltpu.sync_copy(rows0.at[pl.ds(0, 8)],
                        accA.at[pl.ds(_NSUB * _SPT, 8)])
        pltpu.sync_copy(rows0.at[pl.ds(0, 8)],
                        accB.at[pl.ds(_NSUB * _SPT, 8)])

    plsc.subcore_barrier()

    pltpu.sync_copy(idsA_hbm.at[g], idsA_v)
    pltpu.sync_copy(segA_hbm.at[g], segA_v)
    pltpu.sync_copy(idsB_hbm.at[g], idsB_v)
    pltpu.sync_copy(segB_hbm.at[g], segB_v)

    plan = ([(idsA_v, segA_v, accA, j) for j in range(_NCHA)]
            + [(idsB_v, segB_v, accB, j) for j in range(_NCHB)])
    bufs = (rows0, rows1)
    sems = (sem0, sem1)
    handles = [None, None]

    ids0, _, _, j0 = plan[0]
    handles[0] = pltpu.async_copy(emb_hbm.at[ids0.at[j0]], bufs[0], sems[0])
    for k in range(len(plan)):
        if k + 1 < len(plan):
            ids1, _, _, j1 = plan[k + 1]
            handles[(k + 1) % 2] = pltpu.async_copy(
                emb_hbm.at[ids1.at[j1]], bufs[(k + 1) % 2], sems[(k + 1) % 2])
        handles[k % 2].wait()
        _, seg, acc, j = plan[k]
        pltpu.sync_copy(bufs[k % 2], acc.at[seg.at[j]], add=True)

    plsc.subcore_barrier()

    pltpu.sync_copy(accA.at[pl.ds(s * _SPT, _SPT)], rows0.at[pl.ds(0, _SPT)])
    pltpu.sync_copy(accB.at[pl.ds(s * _SPT, _SPT)], rows1.at[pl.ds(0, _SPT)])
    pltpu.sync_copy(rows0.at[pl.ds(0, _SPT)], sumA_hbm.at[pl.ds(base, _SPT)])
    pltpu.sync_copy(rows1.at[pl.ds(0, _SPT)], sumB_hbm.at[pl.ds(base, _SPT)])


def _get_sc_bag():
    if "sc_bag" not in _consts:
        _consts["sc_bag"] = functools.partial(
            pl.kernel,
            out_type=[
                jax.ShapeDtypeStruct((_B, _D), jnp.float32),
                jax.ShapeDtypeStruct((_B, _D), jnp.float32),
            ],
            mesh=plsc.VectorSubcoreMesh(core_axis_name="c", subcore_axis_name="s"),
            compiler_params=pltpu.CompilerParams(use_tc_tiling_on_sc=False),
            scratch_types=[
                pltpu.VMEM((_NCHA, _CHUNK), jnp.int32),
                pltpu.VMEM((_NCHA, _CHUNK), jnp.int32),
                pltpu.VMEM((_NCHB, _CHUNK), jnp.int32),
                pltpu.VMEM((_NCHB, _CHUNK), jnp.int32),
                pltpu.VMEM((_CHUNK, _D), jnp.float32),
                pltpu.VMEM((_CHUNK, _D), jnp.float32),
                pltpu.VMEM_SHARED((_ACC_ROWS, _D), jnp.float32),
                pltpu.VMEM_SHARED((_ACC_ROWS, _D), jnp.float32),
                pltpu.SemaphoreType.DMA,
                pltpu.SemaphoreType.DMA,
            ],
        )(_sc_bag_body)
    return _consts["sc_bag"]


def _tc_loss_body(sumA_ref, sumB_ref, nmask_ref, len_ref, sublen_ref,
                  maskrow_ref, out_ref):
    ri = (sumA_ref[...] + nmask_ref[...] * maskrow_ref[...]) / len_ref[...]
    rj = sumB_ref[...] / sublen_ref[...]
    dn = (((1,), (1,)), ((), ()))
    gij = lax.dot_general(ri, rj, dn, preferred_element_type=jnp.float32,
                          precision=lax.Precision.HIGHEST)
    gii = lax.dot_general(ri, ri, dn, preferred_element_type=jnp.float32,
                          precision=lax.Precision.HIGHEST)
    row = lax.broadcasted_iota(jnp.int32, (_B, _B), 0)
    col = lax.broadcasted_iota(jnp.int32, (_B, _B), 1)
    eye = row == col
    gii = jnp.where(eye, jnp.float32(-1e30), gii)
    m = jnp.maximum(jnp.max(gij, axis=1, keepdims=True),
                    jnp.max(gii, axis=1, keepdims=True))
    se = (jnp.sum(jnp.exp(gij - m), axis=1, keepdims=True)
          + jnp.sum(jnp.exp(gii - m), axis=1, keepdims=True))
    lse = m + jnp.log(se)
    diag = jnp.sum(jnp.where(eye, gij, 0.0), axis=1, keepdims=True)
    out_ref[...] = jnp.reshape(jnp.sum(lse - diag) / _B, (1, 1))


def _tc_loss(sumA, sumB, nmask, lens, sublens, maskrow):
    out = pl.pallas_call(
        _tc_loss_body,
        out_shape=jax.ShapeDtypeStruct((1, 1), jnp.float32),
    )(sumA, sumB, nmask, lens, sublens, maskrow)
    return out[0, 0]


def kernel(sequences, seq_lens, emb):
    P, cumkeep, u = _aug_consts()
    seq_lens = seq_lens.astype(jnp.int32)
    n_keep = jnp.take_along_axis(cumkeep, (seq_lens - 1)[:, None], axis=1)[:, 0]
    n_mask = seq_lens - n_keep
    idsA = jnp.take_along_axis(sequences, P, axis=1)
    lens_f = seq_lens.astype(jnp.float32)
    sub_len = jnp.maximum(1, (_TAO * lens_f).astype(jnp.int32))
    start = (u * (seq_lens - sub_len + 1).astype(jnp.float32)).astype(jnp.int32)
    posB = jnp.clip(start[:, None] + jnp.arange(_KB, dtype=jnp.int32)[None, :],
                    0, _L - 1)
    idsB = jnp.take_along_axis(sequences, posB, axis=1)

    local = (jnp.arange(_B, dtype=jnp.int32) % (_NSUB * _SPT))[:, None]
    segA = jnp.where(jnp.arange(_KA, dtype=jnp.int32)[None, :] < n_keep[:, None],
                     local, _TRASH)
    segB = jnp.where(jnp.arange(_KB, dtype=jnp.int32)[None, :] < sub_len[:, None],
                     local, _TRASH)

    sumA, sumB = _get_sc_bag()(
        idsA.reshape(_NTILES, _NCHA, _CHUNK), segA.reshape(_NTILES, _NCHA, _CHUNK),
        idsB.reshape(_NTILES, _NCHB, _CHUNK), segB.reshape(_NTILES, _NCHB, _CHUNK),
        emb)

    maskrow = lax.slice(emb, (_MASK_ID, 0), (_MASK_ID + 1, _D))
    loss = _tc_loss(sumA, sumB,
                    n_mask.astype(jnp.float32)[:, None],
                    lens_f[:, None],
                    sub_len.astype(jnp.float32)[:, None],
                    maskrow)
    return loss


# speedup vs baseline: 9.7500x; 1.2064x over previous
import functools

import numpy as np
import jax
import jax.numpy as jnp
from jax import lax
from jax.experimental import pallas as pl
from jax.experimental.pallas import tpu as pltpu
from jax.experimental.pallas import tpu_sc as plsc

_B = 1024
_L = 200
_VOCAB = 100000
_D = 64
_MASK_ID = _VOCAB
_GAMMA = 0.7
_TAO = 0.2

_KA = 96
_KB = 48
_CHUNK = 128
_NCORES = 2
_NSUB = 16
_NTILES = _NCORES * _NSUB
_SPT = _B // _NTILES
_ACC_ROWS = _NSUB * _SPT + 8
_TRASH = _NSUB * _SPT

_consts = {}


def _tf_round(x0, x1, r):
    x0 = (x0 + x1) & np.uint64(0xFFFFFFFF)
    x1 = ((x1 << np.uint64(r)) | (x1 >> np.uint64(32 - r))) & np.uint64(0xFFFFFFFF)
    x1 = x0 ^ x1
    return x0, x1


def _threefry2x32(k1, k2, x0, x1):
    rot0 = (13, 15, 26, 6)
    rot1 = (17, 29, 16, 24)
    M = np.uint64(0xFFFFFFFF)
    ks0, ks1 = np.uint64(k1), np.uint64(k2)
    ks2 = ks0 ^ ks1 ^ np.uint64(0x1BD11BDA)
    x0 = (x0 + ks0) & M
    x1 = (x1 + ks1) & M
    for r in rot0:
        x0, x1 = _tf_round(x0, x1, r)
    x0 = (x0 + ks1) & M
    x1 = (x1 + ks2 + np.uint64(1)) & M
    for r in rot1:
        x0, x1 = _tf_round(x0, x1, r)
    x0 = (x0 + ks2) & M
    x1 = (x1 + ks0 + np.uint64(2)) & M
    for r in rot0:
        x0, x1 = _tf_round(x0, x1, r)
    x0 = (x0 + ks0) & M
    x1 = (x1 + ks1 + np.uint64(3)) & M
    for r in rot1:
        x0, x1 = _tf_round(x0, x1, r)
    x0 = (x0 + ks1) & M
    x1 = (x1 + ks2 + np.uint64(4)) & M
    for r in rot0:
        x0, x1 = _tf_round(x0, x1, r)
    x0 = (x0 + ks2) & M
    x1 = (x1 + ks0 + np.uint64(5)) & M
    return x0, x1


def _np_uniform(k1, k2, shape):
    n = int(np.prod(shape))
    b1, b2 = _threefry2x32(k1, k2, np.zeros(n, np.uint64),
                           np.arange(n, dtype=np.uint64))
    bits = (b1 ^ b2).astype(np.uint32)
    fb = (bits >> np.uint32(9)) | np.uint32(0x3F800000)
    return (fb.view(np.float32) - np.float32(1.0)).reshape(shape)


def _aug_consts():
    if "P" not in _consts:
        b1, b2 = _threefry2x32(np.uint64(0), np.uint64(42),
                               np.zeros(2, np.uint64),
                               np.arange(2, dtype=np.uint64))
        bern = _np_uniform(b1[0], b2[0], (_B, _L)) < np.float32(_GAMMA)
        u = _np_uniform(b1[1], b2[1], (_B,))
        keep = ~bern
        cumkeep = np.cumsum(keep, axis=1).astype(np.int32)
        P = np.zeros((_B, _KA), np.int32)
        for b in range(_B):
            pos = np.nonzero(keep[b])[0]
            P[b, : len(pos)] = pos
        local = (np.arange(_B, dtype=np.int32) % _SPT)[:, None]
        PA = (P + _L * local).reshape(_NTILES, _NCHA, _CHUNK)
        _consts["PA"] = jnp.asarray(PA)
        _consts["ckflat"] = jnp.asarray(cumkeep.reshape(-1))
        _consts["u"] = jnp.asarray(u.astype(np.float32))
    return _consts["PA"], _consts["ckflat"], _consts["u"]


_NCHA = _SPT * _KA // _CHUNK
_NCHB = _SPT * _KB // _CHUNK


def _sc_bag_body(seq_hbm, ck_hbm, pa_hbm, len_hbm, start_hbm, sublen_hbm,
                 emb_hbm,
                 sumA_hbm, sumB_hbm, nkeep_hbm,
                 seq_v, ck_v, pa_v, len_v, start_v, sublen_v, nk_v,
                 idsA_v, segA_v, idsB_v, segB_v,
                 rows0, rows1, accA, accB, sem0, sem1, sem2, sem3):
    c = lax.axis_index("c")
    s = lax.axis_index("s")
    g = c * _NSUB + s
    base = g * _SPT

    zv = jnp.zeros((16,), jnp.float32)

    def zero_row(i, carry):
        for k in range(_D // 16):
            rows0[i, pl.ds(k * 16, 16)] = zv
        return carry

    lax.fori_loop(0, _SPT, zero_row, 0)
    pltpu.sync_copy(rows0.at[pl.ds(0, _SPT)], accA.at[pl.ds(s * _SPT, _SPT)])
    pltpu.sync_copy(rows0.at[pl.ds(0, _SPT)], accB.at[pl.ds(s * _SPT, _SPT)])

    @pl.when(s == 0)
    def _():
        pltpu.sync_copy(rows0.at[pl.ds(0, 8)],
                        accA.at[pl.ds(_NSUB * _SPT, 8)])
        pltpu.sync_copy(rows0.at[pl.ds(0, 8)],
                        accB.at[pl.ds(_NSUB * _SPT, 8)])

    plsc.subcore_barrier()

    pltpu.sync_copy(seq_hbm.at[pl.ds(base * _L, _SPT * _L)], seq_v)
    pltpu.sync_copy(ck_hbm.at[pl.ds(base * _L, _SPT * _L)], ck_v)
    pltpu.sync_copy(pa_hbm.at[g], pa_v)
    pltpu.sync_copy(len_hbm.at[pl.ds(base, _SPT)], len_v)
    pltpu.sync_copy(start_hbm.at[pl.ds(base, _SPT)], start_v)
    pltpu.sync_copy(sublen_hbm.at[pl.ds(base, _SPT)], sublen_v)

    iota = lax.iota(jnp.int32, 16)

    for h in range(_SPT // 16):
        bv = h * 16 + iota
        lenv = len_v[pl.ds(h * 16, 16)]
        nk = plsc.load_gather(ck_v, [bv * _L + lenv - 1])
        nk_v[pl.ds(h * 16, 16)] = nk
    pltpu.sync_copy(nk_v, nkeep_hbm.at[pl.ds(base, _SPT)])

    def build_a(j, carry):
        for s16 in range(_CHUNK // 16):
            lvec = j * _CHUNK + s16 * 16 + iota
            bvec = lvec // _KA
            kvec = lvec - bvec * _KA
            pav = pa_v[j, pl.ds(s16 * 16, 16)]
            ids = plsc.load_gather(seq_v, [pav])
            nkb = plsc.load_gather(nk_v, [bvec])
            seg = jnp.where(kvec < nkb, s * _SPT + bvec, _TRASH)
            idsA_v[j, pl.ds(s16 * 16, 16)] = ids
            segA_v[j, pl.ds(s16 * 16, 16)] = seg
        return carry

    lax.fori_loop(0, _NCHA, build_a, 0)

    def build_b(j, carry):
        for s16 in range(_CHUNK // 16):
            lvec = j * _CHUNK + s16 * 16 + iota
            bvec = lvec // _KB
            kvec = lvec - bvec * _KB
            st = plsc.load_gather(start_v, [bvec])
            sl = plsc.load_gather(sublen_v, [bvec])
            pos = jnp.minimum(st + kvec, _L - 1)
            ids = plsc.load_gather(seq_v, [bvec * _L + pos])
            seg = jnp.where(kvec < sl, s * _SPT + bvec, _TRASH)
            idsB_v[j, pl.ds(s16 * 16, 16)] = ids
            segB_v[j, pl.ds(s16 * 16, 16)] = seg
        return carry

    lax.fori_loop(0, _NCHB, build_b, 0)

    plan = ([(idsA_v, segA_v, accA, j) for j in range(_NCHA)]
            + [(idsB_v, segB_v, accB, j) for j in range(_NCHB)])
    n = len(plan)
    bufs = (rows0, rows1)
    gsems = (sem0, sem1)
    ssems = (sem2, sem3)
    ghandles = [None, None]
    shandles = [None, None]

    ids0, _, _, j0 = plan[0]
    ghandles[0] = pltpu.async_copy(emb_hbm.at[ids0.at[j0]], bufs[0], gsems[0])
    for k in range(n):
        if k + 1 < n:
            if shandles[(k + 1) % 2] is not None:
                shandles[(k + 1) % 2].wait()
                shandles[(k + 1) % 2] = None
            ids1, _, _, j1 = plan[k + 1]
            ghandles[(k + 1) % 2] = pltpu.async_copy(
                emb_hbm.at[ids1.at[j1]], bufs[(k + 1) % 2], gsems[(k + 1) % 2])
        ghandles[k % 2].wait()
        _, seg, acc, j = plan[k]
        shandles[k % 2] = pltpu.async_copy(
            bufs[k % 2], acc.at[seg.at[j]], ssems[k % 2], add=True)
    for h in shandles:
        if h is not None:
            h.wait()

    plsc.subcore_barrier()

    pltpu.sync_copy(accA.at[pl.ds(s * _SPT, _SPT)], rows0.at[pl.ds(0, _SPT)])
    pltpu.sync_copy(accB.at[pl.ds(s * _SPT, _SPT)], rows1.at[pl.ds(0, _SPT)])
    pltpu.sync_copy(rows0.at[pl.ds(0, _SPT)], sumA_hbm.at[pl.ds(base, _SPT)])
    pltpu.sync_copy(rows1.at[pl.ds(0, _SPT)], sumB_hbm.at[pl.ds(base, _SPT)])


def _get_sc_bag():
    if "sc_bag" not in _consts:
        _consts["sc_bag"] = functools.partial(
            pl.kernel,
            out_type=[
                jax.ShapeDtypeStruct((_B, _D), jnp.float32),
                jax.ShapeDtypeStruct((_B, _D), jnp.float32),
                jax.ShapeDtypeStruct((_B,), jnp.int32),
            ],
            mesh=plsc.VectorSubcoreMesh(core_axis_name="c", subcore_axis_name="s"),
            compiler_params=pltpu.CompilerParams(
                use_tc_tiling_on_sc=False, needs_layout_passes=False),
            scratch_types=[
                pltpu.VMEM((_SPT * _L,), jnp.int32),
                pltpu.VMEM((_SPT * _L,), jnp.int32),
                pltpu.VMEM((_NCHA, _CHUNK), jnp.int32),
                pltpu.VMEM((_SPT,), jnp.int32),
                pltpu.VMEM((_SPT,), jnp.int32),
                pltpu.VMEM((_SPT,), jnp.int32),
                pltpu.VMEM((_SPT,), jnp.int32),
                pltpu.VMEM((_NCHA, _CHUNK), jnp.int32),
                pltpu.VMEM((_NCHA, _CHUNK), jnp.int32),
                pltpu.VMEM((_NCHB, _CHUNK), jnp.int32),
                pltpu.VMEM((_NCHB, _CHUNK), jnp.int32),
                pltpu.VMEM((_CHUNK, _D), jnp.float32),
                pltpu.VMEM((_CHUNK, _D), jnp.float32),
                pltpu.VMEM_SHARED((_ACC_ROWS, _D), jnp.float32),
                pltpu.VMEM_SHARED((_ACC_ROWS, _D), jnp.float32),
                pltpu.SemaphoreType.DMA,
                pltpu.SemaphoreType.DMA,
                pltpu.SemaphoreType.DMA,
                pltpu.SemaphoreType.DMA,
            ],
        )(_sc_bag_body)
    return _consts["sc_bag"]


def _tc_loss_body(sumA_ref, sumB_ref, nmask_ref, len_ref, sublen_ref,
                  maskrow_ref, out_ref):
    ri = (sumA_ref[...] + nmask_ref[...] * maskrow_ref[...]) / len_ref[...]
    rj = sumB_ref[...] / sublen_ref[...]
    dn = (((1,), (1,)), ((), ()))
    gij = lax.dot_general(ri, rj, dn, preferred_element_type=jnp.float32,
                          precision=lax.Precision.HIGHEST)
    gii = lax.dot_general(ri, ri, dn, preferred_element_type=jnp.float32,
                          precision=lax.Precision.HIGHEST)
    row = lax.broadcasted_iota(jnp.int32, (_B, _B), 0)
    col = lax.broadcasted_iota(jnp.int32, (_B, _B), 1)
    eye = row == col
    gii = jnp.where(eye, jnp.float32(-1e30), gii)
    m = jnp.maximum(jnp.max(gij, axis=1, keepdims=True),
                    jnp.max(gii, axis=1, keepdims=True))
    se = (jnp.sum(jnp.exp(gij - m), axis=1, keepdims=True)
          + jnp.sum(jnp.exp(gii - m), axis=1, keepdims=True))
    lse = m + jnp.log(se)
    diag = jnp.sum(jnp.where(eye, gij, 0.0), axis=1, keepdims=True)
    out_ref[...] = jnp.reshape(jnp.sum(lse - diag) / _B, (1, 1))


def _tc_loss(sumA, sumB, nmask, lens, sublens, maskrow):
    out = pl.pallas_call(
        _tc_loss_body,
        out_shape=jax.ShapeDtypeStruct((1, 1), jnp.float32),
    )(sumA, sumB, nmask, lens, sublens, maskrow)
    return out[0, 0]


def kernel(sequences, seq_lens, emb):
    PA, ckflat, u = _aug_consts()
    seq_lens = seq_lens.astype(jnp.int32)
    lens_f = seq_lens.astype(jnp.float32)
    sub_len = jnp.maximum(1, (_TAO * lens_f).astype(jnp.int32))
    start = (u * (seq_lens - sub_len + 1).astype(jnp.float32)).astype(jnp.int32)

    sumA, sumB, n_keep = _get_sc_bag()(
        sequences.reshape(-1), ckflat, PA, seq_lens, start, sub_len, emb)

    n_mask = (seq_lens - n_keep).astype(jnp.float32)
    maskrow = lax.slice(emb, (_MASK_ID, 0), (_MASK_ID + 1, _D))
    loss = _tc_loss(sumA, sumB,
                    n_mask[:, None],
                    lens_f[:, None],
                    sub_len.astype(jnp.float32)[:, None],
                    maskrow)
    return loss
